# baseline (speedup 1.0000x reference)
import functools

import jax
import jax.numpy as jnp
from jax import lax
from jax.experimental import pallas as pl
from jax.experimental.pallas import tpu as pltpu
from jax.experimental.pallas import tpu_sc as plsc

ACTIVE = 256
_NC, _NS = 2, 16


def _p1_body(x_ref, wsalt_ref, sal_ref, sum_ref):
    l = pl.program_id(1)
    xb = x_ref[0]
    sal_ref[0] = jnp.dot(xb, wsalt_ref[...], preferred_element_type=jnp.float32)
    part = jnp.sum(xb, axis=0, keepdims=True)

    @pl.when(l == 0)
    def _init():
        sum_ref[0] = part

    @pl.when(l > 0)
    def _acc():
        sum_ref[0] = sum_ref[0] + part


def _pass1(x, w_sal_t, BL):
    B, L, D = x.shape
    return pl.pallas_call(
        _p1_body,
        grid=(B, L // BL),
        in_specs=[
            pl.BlockSpec((1, BL, D), lambda b, l: (b, l, 0)),
            pl.BlockSpec((D, 1), lambda b, l: (0, 0)),
        ],
        out_specs=[
            pl.BlockSpec((1, BL, 1), lambda b, l: (b, l, 0)),
            pl.BlockSpec((1, 1, D), lambda b, l: (b, 0, 0)),
        ],
        out_shape=[
            jax.ShapeDtypeStruct((B, L, 1), jnp.float32),
            jax.ShapeDtypeStruct((B, 1, D), jnp.float32),
        ],
    )(x, w_sal_t)


def _as_i32(t_u32):
    return lax.bitcast_convert_type(t_u32 ^ jnp.uint32(0x80000000), jnp.int32)


def _count_gt(ref, nv, t_i32):
    def step(j, acc):
        return acc + jnp.where(ref[pl.ds(j * 16, 16)] > t_i32, 1, 0)

    acc = lax.fori_loop(0, nv, step, jnp.zeros((16,), jnp.int32), unroll=8)
    return jnp.sum(acc)


def _find_thr(ref, nv, keep):
    def bstep(_, lohi):
        lo, hi = lohi
        mid = lo + ((hi - lo) >> jnp.uint32(1))
        le = _count_gt(ref, nv, _as_i32(mid)) <= (keep - 1)
        return (jnp.where(le, lo, mid + jnp.uint32(1)),
                jnp.where(le, mid, hi))

    lo, _ = lax.fori_loop(
        0, 32, bstep, (jnp.uint32(0), jnp.uint32(0xFFFFFFFF)))
    return _as_i32(lo)


def _sc_select_gather(sal_bits, x2d, B, L, D):
    parts = 8
    npp = ACTIVE // parts
    SL = L // parts
    nvs = SL // 16
    ncand = parts * ACTIVE
    nvc = ncand // 16

    mesh = plsc.VectorSubcoreMesh(
        core_axis_name="c", subcore_axis_name="s",
        num_cores=_NC, num_subcores=_NS)

    @functools.partial(
        pl.kernel,
        out_type=[
            jax.ShapeDtypeStruct((B * ACTIVE, D), jnp.float32),
            jax.ShapeDtypeStruct((B * parts, D), jnp.float32),
        ],
        mesh=mesh,
        scratch_types=[
            pltpu.VMEM((SL,), jnp.int32),
            pltpu.VMEM((ACTIVE,), jnp.int32),
            pltpu.VMEM((ACTIVE,), jnp.int32),
            pltpu.VMEM((ncand,), jnp.int32),
            pltpu.VMEM((ncand,), jnp.int32),
            pltpu.VMEM((ACTIVE,), jnp.int32),
            pltpu.VMEM((npp,), jnp.int32),
            pltpu.VMEM((npp, D), jnp.float32),
            pltpu.VMEM((1, D), jnp.float32),
            pltpu.VMEM_SHARED((2 * ncand,), jnp.int32),
            pltpu.VMEM_SHARED((2 * ncand,), jnp.int32),
            pltpu.VMEM_SHARED((2 * ACTIVE,), jnp.int32),
            pltpu.SemaphoreType.DMA,
        ],
        compiler_params=pltpu.CompilerParams(needs_layout_passes=False),
    )
    def k(sal_hbm, x_hbm, act_hbm, psum_hbm,
          key_v, ckey_v, cidx_v, lk_v, li_v, fidx_v, myidx_v, rows_v, acc_v,
          skey_s, sidx_s, sfidx_s, sem):
        c = lax.axis_index("c")
        s = lax.axis_index("s")
        g = s // parts
        b = c * 2 + g
        p = s % parts

        base = b * L + p * SL
        pltpu.sync_copy(sal_hbm.at[pl.ds(base, SL)], key_v)

        minint = jnp.int32(-2147483648)

        def build(j, _):
            bits = key_v[pl.ds(j * 16, 16)]
            bits = jnp.where(bits == minint, 0, bits)
            key_v[pl.ds(j * 16, 16)] = jnp.where(
                bits < 0, ~bits ^ minint, bits)
            return 0

        lax.fori_loop(0, nvs, build, 0, unroll=8)

        lthr = _find_thr(key_v, nvs, ACTIVE)
        lr_eq = ACTIVE - _count_gt(key_v, nvs, lthr)

        def lcomp(j, carry):
            kcnt, eqseen = carry
            kv = key_v[pl.ds(j * 16, 16)]
            gt = kv > lthr
            eq = kv == lthr
            eqc = plsc.cumsum(jnp.where(eq, 1, 0))
            keep = gt | (eq & ((eqseen + eqc) <= lr_eq))
            pos = kcnt + plsc.cumsum(jnp.where(keep, 1, 0)) - 1
            pos = jnp.where(keep, pos, 0)
            gidx = lax.iota(jnp.int32, 16) + (j * 16 + base)
            plsc.store_scatter(cidx_v, [pos], gidx, mask=keep)
            plsc.store_scatter(ckey_v, [pos], kv, mask=keep)
            return (kcnt + jnp.sum(jnp.where(keep, 1, 0)),
                    eqseen + jnp.sum(jnp.where(eq, 1, 0)))

        lax.fori_loop(0, nvs, lcomp, (jnp.int32(0), jnp.int32(0)), unroll=2)

        slot = g * ncand + p * ACTIVE
        pltpu.sync_copy(ckey_v, skey_s.at[pl.ds(slot, ACTIVE)])
        pltpu.sync_copy(cidx_v, sidx_s.at[pl.ds(slot, ACTIVE)])
        plsc.subcore_barrier()

        @pl.when(p == 0)
        def _select():
            pltpu.sync_copy(skey_s.at[pl.ds(g * ncand, ncand)], lk_v)
            pltpu.sync_copy(sidx_s.at[pl.ds(g * ncand, ncand)], li_v)
            thr = _find_thr(lk_v, nvc, ACTIVE)
            r_eq = ACTIVE - _count_gt(lk_v, nvc, thr)

            def fcomp(j, carry):
                kcnt, eqseen = carry
                kv = lk_v[pl.ds(j * 16, 16)]
                iv = li_v[pl.ds(j * 16, 16)]
                gt = kv > thr
                eq = kv == thr
                eqc = plsc.cumsum(jnp.where(eq, 1, 0))
                keep = gt | (eq & ((eqseen + eqc) <= r_eq))
                pos = kcnt + plsc.cumsum(jnp.where(keep, 1, 0)) - 1
                pos = jnp.where(keep, pos, 0)
                plsc.store_scatter(fidx_v, [pos], iv, mask=keep)
                return (kcnt + jnp.sum(jnp.where(keep, 1, 0)),
                        eqseen + jnp.sum(jnp.where(eq, 1, 0)))

            lax.fori_loop(0, nvc, fcomp, (jnp.int32(0), jnp.int32(0)),
                          unroll=2)
            pltpu.sync_copy(fidx_v, sfidx_s.at[pl.ds(g * ACTIVE, ACTIVE)])

        plsc.subcore_barrier()

        row0 = b * ACTIVE + p * npp
        pltpu.sync_copy(sfidx_s.at[pl.ds(g * ACTIVE + p * npp, npp)], myidx_v)
        pltpu.async_copy(x_hbm.at[myidx_v], rows_v, sem).wait()
        pltpu.sync_copy(rows_v, act_hbm.at[pl.ds(row0, npp)])

        def colsum(ccol, _):
            def rstep(rr, acc):
                return acc + rows_v[rr, pl.ds(ccol * 16, 16)]

            acc_v[0, pl.ds(ccol * 16, 16)] = lax.fori_loop(
                0, npp, rstep, jnp.zeros((16,), jnp.float32), unroll=8)
            return 0

        lax.fori_loop(0, D // 16, colsum, 0, unroll=2)
        pltpu.sync_copy(acc_v, psum_hbm.at[pl.ds(b * parts + p, 1)])

    return k(sal_bits, x2d)


def _p3_body(x_ref, wread_ref, sumall_ref, psum_ref, out_ref, cold_ref, v_scr):
    l = pl.program_id(1)

    @pl.when(l == 0)
    def _ctx():
        sk = jnp.sum(psum_ref[0], axis=0, keepdims=True)
        n_drop = x_ref.shape[1] * pl.num_programs(1) - ACTIVE
        cold = (sumall_ref[0] - sk) / float(n_drop)
        ctx = sk * (1.0 / ACTIVE) + cold
        v_scr[...] = lax.dot_general(
            ctx, wread_ref[...], (((1,), (1,)), ((), ())),
            preferred_element_type=jnp.float32)
        cold_ref[0] = cold

    out_ref[0] = x_ref[0] + v_scr[...]


def _pass3(x, w_read, sum_all, psum, BL):
    B, L, D = x.shape
    return pl.pallas_call(
        _p3_body,
        grid=(B, L // BL),
        in_specs=[
            pl.BlockSpec((1, BL, D), lambda b, l: (b, l, 0)),
            pl.BlockSpec((D, D), lambda b, l: (0, 0)),
            pl.BlockSpec((1, 1, D), lambda b, l: (b, 0, 0)),
            pl.BlockSpec((1, 8, D), lambda b, l: (b, 0, 0)),
        ],
        out_specs=[
            pl.BlockSpec((1, BL, D), lambda b, l: (b, l, 0)),
            pl.BlockSpec((1, 1, D), lambda b, l: (b, 0, 0)),
        ],
        out_shape=[
            jax.ShapeDtypeStruct((B, L, D), jnp.float32),
            jax.ShapeDtypeStruct((B, 1, D), jnp.float32),
        ],
        scratch_shapes=[pltpu.VMEM((1, D), jnp.float32)],
    )(x, w_read, sum_all, psum)


def kernel(x, w_sal, w_read):
    B, L, D = x.shape
    sal3, sum_all = _pass1(x, w_sal.T, 4096)
    sal_bits = lax.bitcast_convert_type(sal3.reshape(B * L), jnp.int32)
    act2d, psum2d = _sc_select_gather(sal_bits, x.reshape(B * L, D), B, L, D)
    x_out, cold3 = _pass3(x, w_read, sum_all, psum2d.reshape(B, 8, D), 2048)
    return x_out, act2d.reshape(B, ACTIVE, D), cold3.reshape(B, D)


# speedup vs baseline: 1.0113x; 1.0113x over previous
import functools

import jax
import jax.numpy as jnp
from jax import lax
from jax.experimental import pallas as pl
from jax.experimental.pallas import tpu as pltpu
from jax.experimental.pallas import tpu_sc as plsc

ACTIVE = 256
_NC, _NS = 2, 16


def _p1_body(x_ref, wsalt_ref, sal_ref, sum_ref):
    l = pl.program_id(1)
    xb = x_ref[0]
    sal_ref[0] = jnp.dot(xb, wsalt_ref[...], preferred_element_type=jnp.float32)
    part = jnp.sum(xb, axis=0, keepdims=True)

    @pl.when(l == 0)
    def _init():
        sum_ref[0] = part

    @pl.when(l > 0)
    def _acc():
        sum_ref[0] = sum_ref[0] + part


def _pass1(x, w_sal_t, BL):
    B, L, D = x.shape
    return pl.pallas_call(
        _p1_body,
        grid=(B, L // BL),
        in_specs=[
            pl.BlockSpec((1, BL, D), lambda b, l: (b, l, 0)),
            pl.BlockSpec((D, 1), lambda b, l: (0, 0)),
        ],
        out_specs=[
            pl.BlockSpec((1, BL, 1), lambda b, l: (b, l, 0)),
            pl.BlockSpec((1, 1, D), lambda b, l: (b, 0, 0)),
        ],
        out_shape=[
            jax.ShapeDtypeStruct((B, L, 1), jnp.float32),
            jax.ShapeDtypeStruct((B, 1, D), jnp.float32),
        ],
    )(x, w_sal_t)


def _as_i32(t_u32):
    return lax.bitcast_convert_type(t_u32 ^ jnp.uint32(0x80000000), jnp.int32)


def _count_gt(ref, nv, t_i32):
    def step(j, acc):
        return acc + jnp.where(ref[pl.ds(j * 16, 16)] > t_i32, 1, 0)

    acc = lax.fori_loop(0, nv, step, jnp.zeros((16,), jnp.int32), unroll=8)
    return jnp.sum(acc)


def _find_thr(ref, nv, keep):
    def bstep(_, lohi):
        lo, hi = lohi
        mid = lo + ((hi - lo) >> jnp.uint32(1))
        le = _count_gt(ref, nv, _as_i32(mid)) <= (keep - 1)
        return (jnp.where(le, lo, mid + jnp.uint32(1)),
                jnp.where(le, mid, hi))

    lo, _ = lax.fori_loop(
        0, 32, bstep, (jnp.uint32(0), jnp.uint32(0xFFFFFFFF)))
    return _as_i32(lo)


def _sc_select_gather(sal_bits, x2d, B, L, D):
    parts = 8
    npp = ACTIVE // parts
    SL = L // parts
    nvs = SL // 16
    ncand = parts * ACTIVE
    nvc = ncand // 16

    mesh = plsc.VectorSubcoreMesh(
        core_axis_name="c", subcore_axis_name="s",
        num_cores=_NC, num_subcores=_NS)

    @functools.partial(
        pl.kernel,
        out_type=[
            jax.ShapeDtypeStruct((B * ACTIVE, D), jnp.float32),
            jax.ShapeDtypeStruct((B * parts, D), jnp.float32),
        ],
        mesh=mesh,
        scratch_types=[
            pltpu.VMEM((SL,), jnp.int32),
            pltpu.VMEM((ACTIVE,), jnp.int32),
            pltpu.VMEM((ACTIVE,), jnp.int32),
            pltpu.VMEM((ncand,), jnp.int32),
            pltpu.VMEM((ncand,), jnp.int32),
            pltpu.VMEM((ACTIVE,), jnp.int32),
            pltpu.VMEM((npp,), jnp.int32),
            pltpu.VMEM((npp, D), jnp.float32),
            pltpu.VMEM((1, D), jnp.float32),
            pltpu.VMEM_SHARED((2 * ncand,), jnp.int32),
            pltpu.VMEM_SHARED((2 * ncand,), jnp.int32),
            pltpu.VMEM_SHARED((2 * ACTIVE,), jnp.int32),
            pltpu.SemaphoreType.DMA,
        ],
        compiler_params=pltpu.CompilerParams(needs_layout_passes=False),
    )
    def k(sal_hbm, x_hbm, act_hbm, psum_hbm,
          key_v, ckey_v, cidx_v, lk_v, li_v, fidx_v, myidx_v, rows_v, acc_v,
          skey_s, sidx_s, sfidx_s, sem):
        c = lax.axis_index("c")
        s = lax.axis_index("s")
        g = s // parts
        b = c * 2 + g
        p = s % parts

        base = b * L + p * SL
        pltpu.sync_copy(sal_hbm.at[pl.ds(base, SL)], key_v)

        minint = jnp.int32(-2147483648)

        def build(j, _):
            bits = key_v[pl.ds(j * 16, 16)]
            bits = jnp.where(bits == minint, 0, bits)
            key_v[pl.ds(j * 16, 16)] = jnp.where(
                bits < 0, ~bits ^ minint, bits)
            return 0

        lax.fori_loop(0, nvs, build, 0, unroll=8)

        lthr = _find_thr(key_v, nvs, ACTIVE)
        lr_eq = ACTIVE - _count_gt(key_v, nvs, lthr)

        def lcomp(j, carry):
            kcnt, eqseen = carry
            kv = key_v[pl.ds(j * 16, 16)]
            gt = kv > lthr
            eq = kv == lthr
            eqc = plsc.cumsum(jnp.where(eq, 1, 0))
            keep = gt | (eq & ((eqseen + eqc) <= lr_eq))
            pos = kcnt + plsc.cumsum(jnp.where(keep, 1, 0)) - 1
            pos = jnp.where(keep, pos, 0)
            gidx = lax.iota(jnp.int32, 16) + (j * 16 + base)
            plsc.store_scatter(cidx_v, [pos], gidx, mask=keep)
            plsc.store_scatter(ckey_v, [pos], kv, mask=keep)
            return (kcnt + jnp.sum(jnp.where(keep, 1, 0)),
                    eqseen + jnp.sum(jnp.where(eq, 1, 0)))

        lax.fori_loop(0, nvs, lcomp, (jnp.int32(0), jnp.int32(0)), unroll=2)

        slot = g * ncand + p * ACTIVE
        pltpu.sync_copy(ckey_v, skey_s.at[pl.ds(slot, ACTIVE)])
        pltpu.sync_copy(cidx_v, sidx_s.at[pl.ds(slot, ACTIVE)])
        plsc.subcore_barrier()

        @pl.when(p == 0)
        def _select():
            pltpu.sync_copy(skey_s.at[pl.ds(g * ncand, ncand)], lk_v)
            pltpu.sync_copy(sidx_s.at[pl.ds(g * ncand, ncand)], li_v)
            thr = _find_thr(lk_v, nvc, ACTIVE)
            r_eq = ACTIVE - _count_gt(lk_v, nvc, thr)

            def fcomp(j, carry):
                kcnt, eqseen = carry
                kv = lk_v[pl.ds(j * 16, 16)]
                iv = li_v[pl.ds(j * 16, 16)]
                gt = kv > thr
                eq = kv == thr
                eqc = plsc.cumsum(jnp.where(eq, 1, 0))
                keep = gt | (eq & ((eqseen + eqc) <= r_eq))
                pos = kcnt + plsc.cumsum(jnp.where(keep, 1, 0)) - 1
                pos = jnp.where(keep, pos, 0)
                plsc.store_scatter(fidx_v, [pos], iv, mask=keep)
                return (kcnt + jnp.sum(jnp.where(keep, 1, 0)),
                        eqseen + jnp.sum(jnp.where(eq, 1, 0)))

            lax.fori_loop(0, nvc, fcomp, (jnp.int32(0), jnp.int32(0)),
                          unroll=2)
            pltpu.sync_copy(fidx_v, sfidx_s.at[pl.ds(g * ACTIVE, ACTIVE)])

        plsc.subcore_barrier()

        row0 = b * ACTIVE + p * npp
        pltpu.sync_copy(sfidx_s.at[pl.ds(g * ACTIVE + p * npp, npp)], myidx_v)
        pltpu.async_copy(x_hbm.at[myidx_v], rows_v, sem).wait()
        out_cp = pltpu.make_async_copy(
            rows_v, act_hbm.at[pl.ds(row0, npp)], sem)
        out_cp.start()

        def colsum(ccol, _):
            def rstep(rr, acc):
                return acc + rows_v[rr, pl.ds(ccol * 16, 16)]

            acc_v[0, pl.ds(ccol * 16, 16)] = lax.fori_loop(
                0, npp, rstep, jnp.zeros((16,), jnp.float32), unroll=8)
            return 0

        lax.fori_loop(0, D // 16, colsum, 0, unroll=4)
        pltpu.sync_copy(acc_v, psum_hbm.at[pl.ds(b * parts + p, 1)])
        out_cp.wait()

    return k(sal_bits, x2d)


def _p3_body(x_ref, wread_ref, sumall_ref, psum_ref, out_ref, cold_ref, v_scr):
    l = pl.program_id(1)

    @pl.when(l == 0)
    def _ctx():
        sk = jnp.sum(psum_ref[0], axis=0, keepdims=True)
        n_drop = x_ref.shape[1] * pl.num_programs(1) - ACTIVE
        cold = (sumall_ref[0] - sk) / float(n_drop)
        ctx = sk * (1.0 / ACTIVE) + cold
        v_scr[...] = lax.dot_general(
            ctx, wread_ref[...], (((1,), (1,)), ((), ())),
            preferred_element_type=jnp.float32)
        cold_ref[0] = cold

    out_ref[0] = x_ref[0] + v_scr[...]


def _pass3(x, w_read, sum_all, psum, BL):
    B, L, D = x.shape
    return pl.pallas_call(
        _p3_body,
        grid=(B, L // BL),
        in_specs=[
            pl.BlockSpec((1, BL, D), lambda b, l: (b, l, 0)),
            pl.BlockSpec((D, D), lambda b, l: (0, 0)),
            pl.BlockSpec((1, 1, D), lambda b, l: (b, 0, 0)),
            pl.BlockSpec((1, 8, D), lambda b, l: (b, 0, 0)),
        ],
        out_specs=[
            pl.BlockSpec((1, BL, D), lambda b, l: (b, l, 0)),
            pl.BlockSpec((1, 1, D), lambda b, l: (b, 0, 0)),
        ],
        out_shape=[
            jax.ShapeDtypeStruct((B, L, D), jnp.float32),
            jax.ShapeDtypeStruct((B, 1, D), jnp.float32),
        ],
        scratch_shapes=[pltpu.VMEM((1, D), jnp.float32)],
    )(x, w_read, sum_all, psum)


def kernel(x, w_sal, w_read):
    B, L, D = x.shape
    sal3, sum_all = _pass1(x, w_sal.T, 2048)
    sal_bits = lax.bitcast_convert_type(sal3.reshape(B * L), jnp.int32)
    act2d, psum2d = _sc_select_gather(sal_bits, x.reshape(B * L, D), B, L, D)
    x_out, cold3 = _pass3(x, w_read, sum_all, psum2d.reshape(B, 8, D), 2048)
    return x_out, act2d.reshape(B, ACTIVE, D), cold3.reshape(B, D)


# speedup vs baseline: 1.0152x; 1.0039x over previous
import functools

import jax
import jax.numpy as jnp
from jax import lax
from jax.experimental import pallas as pl
from jax.experimental.pallas import tpu as pltpu
from jax.experimental.pallas import tpu_sc as plsc

ACTIVE = 256
_NC, _NS = 2, 16


def _p1_body(x_ref, wsalt_ref, sal_ref, sum_ref):
    l = pl.program_id(1)
    xb = x_ref[0]
    sal_ref[0] = jnp.dot(xb, wsalt_ref[...], preferred_element_type=jnp.float32)
    part = jnp.sum(xb, axis=0, keepdims=True)

    @pl.when(l == 0)
    def _init():
        sum_ref[0] = part

    @pl.when(l > 0)
    def _acc():
        sum_ref[0] = sum_ref[0] + part


def _pass1(x, w_sal_t, BL):
    B, L, D = x.shape
    return pl.pallas_call(
        _p1_body,
        grid=(B, L // BL),
        in_specs=[
            pl.BlockSpec((1, BL, D), lambda b, l: (b, l, 0)),
            pl.BlockSpec((D, 1), lambda b, l: (0, 0)),
        ],
        out_specs=[
            pl.BlockSpec((1, BL, 1), lambda b, l: (b, l, 0)),
            pl.BlockSpec((1, 1, D), lambda b, l: (b, 0, 0)),
        ],
        out_shape=[
            jax.ShapeDtypeStruct((B, L, 1), jnp.float32),
            jax.ShapeDtypeStruct((B, 1, D), jnp.float32),
        ],
    )(x, w_sal_t)


def _as_i32(t_u32):
    return lax.bitcast_convert_type(t_u32 ^ jnp.uint32(0x80000000), jnp.int32)


def _count_gt(ref, nv, t_i32):
    def step(j, accs):
        a0, a1, a2, a3 = accs
        base = j * 64
        a0 = a0 + jnp.where(ref[pl.ds(base, 16)] > t_i32, 1, 0)
        a1 = a1 + jnp.where(ref[pl.ds(base + 16, 16)] > t_i32, 1, 0)
        a2 = a2 + jnp.where(ref[pl.ds(base + 32, 16)] > t_i32, 1, 0)
        a3 = a3 + jnp.where(ref[pl.ds(base + 48, 16)] > t_i32, 1, 0)
        return (a0, a1, a2, a3)

    z = jnp.zeros((16,), jnp.int32)
    a0, a1, a2, a3 = lax.fori_loop(0, nv // 4, step, (z, z, z, z), unroll=2)
    return jnp.sum((a0 + a1) + (a2 + a3))


def _find_thr(ref, nv, keep):
    def bstep(_, lohi):
        lo, hi = lohi
        mid = lo + ((hi - lo) >> jnp.uint32(1))
        le = _count_gt(ref, nv, _as_i32(mid)) <= (keep - 1)
        return (jnp.where(le, lo, mid + jnp.uint32(1)),
                jnp.where(le, mid, hi))

    lo, _ = lax.fori_loop(
        0, 32, bstep, (jnp.uint32(0), jnp.uint32(0xFFFFFFFF)))
    return _as_i32(lo)


def _sc_select_gather(sal_bits, x2d, B, L, D):
    parts = 8
    npp = ACTIVE // parts
    SL = L // parts
    nvs = SL // 16
    ncand = parts * ACTIVE
    nvc = ncand // 16

    mesh = plsc.VectorSubcoreMesh(
        core_axis_name="c", subcore_axis_name="s",
        num_cores=_NC, num_subcores=_NS)

    @functools.partial(
        pl.kernel,
        out_type=[
            jax.ShapeDtypeStruct((B * ACTIVE, D), jnp.float32),
            jax.ShapeDtypeStruct((B * parts, D), jnp.float32),
        ],
        mesh=mesh,
        scratch_types=[
            pltpu.VMEM((SL,), jnp.int32),
            pltpu.VMEM((ACTIVE,), jnp.int32),
            pltpu.VMEM((ACTIVE,), jnp.int32),
            pltpu.VMEM((ncand,), jnp.int32),
            pltpu.VMEM((ncand,), jnp.int32),
            pltpu.VMEM((ACTIVE,), jnp.int32),
            pltpu.VMEM((npp,), jnp.int32),
            pltpu.VMEM((npp, D), jnp.float32),
            pltpu.VMEM((1, D), jnp.float32),
            pltpu.VMEM_SHARED((2 * ncand,), jnp.int32),
            pltpu.VMEM_SHARED((2 * ncand,), jnp.int32),
            pltpu.VMEM_SHARED((2 * ACTIVE,), jnp.int32),
            pltpu.SemaphoreType.DMA,
        ],
        compiler_params=pltpu.CompilerParams(needs_layout_passes=False),
    )
    def k(sal_hbm, x_hbm, act_hbm, psum_hbm,
          key_v, ckey_v, cidx_v, lk_v, li_v, fidx_v, myidx_v, rows_v, acc_v,
          skey_s, sidx_s, sfidx_s, sem):
        c = lax.axis_index("c")
        s = lax.axis_index("s")
        g = s // parts
        b = c * 2 + g
        p = s % parts

        base = b * L + p * SL
        pltpu.sync_copy(sal_hbm.at[pl.ds(base, SL)], key_v)

        minint = jnp.int32(-2147483648)

        def build(j, _):
            bits = key_v[pl.ds(j * 16, 16)]
            bits = jnp.where(bits == minint, 0, bits)
            key_v[pl.ds(j * 16, 16)] = jnp.where(
                bits < 0, ~bits ^ minint, bits)
            return 0

        lax.fori_loop(0, nvs, build, 0, unroll=8)

        lthr = _find_thr(key_v, nvs, ACTIVE)
        lr_eq = ACTIVE - _count_gt(key_v, nvs, lthr)

        def lcomp(j, carry):
            kcnt, eqseen = carry
            kv = key_v[pl.ds(j * 16, 16)]
            gt = kv > lthr
            eq = kv == lthr
            eqc = plsc.cumsum(jnp.where(eq, 1, 0))
            keep = gt | (eq & ((eqseen + eqc) <= lr_eq))
            pos = kcnt + plsc.cumsum(jnp.where(keep, 1, 0)) - 1
            pos = jnp.where(keep, pos, 0)
            gidx = lax.iota(jnp.int32, 16) + (j * 16 + base)
            plsc.store_scatter(cidx_v, [pos], gidx, mask=keep)
            plsc.store_scatter(ckey_v, [pos], kv, mask=keep)
            return (kcnt + jnp.sum(jnp.where(keep, 1, 0)),
                    eqseen + jnp.sum(jnp.where(eq, 1, 0)))

        lax.fori_loop(0, nvs, lcomp, (jnp.int32(0), jnp.int32(0)), unroll=2)

        slot = g * ncand + p * ACTIVE
        pltpu.sync_copy(ckey_v, skey_s.at[pl.ds(slot, ACTIVE)])
        pltpu.sync_copy(cidx_v, sidx_s.at[pl.ds(slot, ACTIVE)])
        plsc.subcore_barrier()

        @pl.when(p == 0)
        def _select():
            pltpu.sync_copy(skey_s.at[pl.ds(g * ncand, ncand)], lk_v)
            pltpu.sync_copy(sidx_s.at[pl.ds(g * ncand, ncand)], li_v)
            thr = _find_thr(lk_v, nvc, ACTIVE)
            r_eq = ACTIVE - _count_gt(lk_v, nvc, thr)

            def fcomp(j, carry):
                kcnt, eqseen = carry
                kv = lk_v[pl.ds(j * 16, 16)]
                iv = li_v[pl.ds(j * 16, 16)]
                gt = kv > thr
                eq = kv == thr
                eqc = plsc.cumsum(jnp.where(eq, 1, 0))
                keep = gt | (eq & ((eqseen + eqc) <= r_eq))
                pos = kcnt + plsc.cumsum(jnp.where(keep, 1, 0)) - 1
                pos = jnp.where(keep, pos, 0)
                plsc.store_scatter(fidx_v, [pos], iv, mask=keep)
                return (kcnt + jnp.sum(jnp.where(keep, 1, 0)),
                        eqseen + jnp.sum(jnp.where(eq, 1, 0)))

            lax.fori_loop(0, nvc, fcomp, (jnp.int32(0), jnp.int32(0)),
                          unroll=2)
            pltpu.sync_copy(fidx_v, sfidx_s.at[pl.ds(g * ACTIVE, ACTIVE)])

        plsc.subcore_barrier()

        row0 = b * ACTIVE + p * npp
        pltpu.sync_copy(sfidx_s.at[pl.ds(g * ACTIVE + p * npp, npp)], myidx_v)
        pltpu.async_copy(x_hbm.at[myidx_v], rows_v, sem).wait()
        out_cp = pltpu.make_async_copy(
            rows_v, act_hbm.at[pl.ds(row0, npp)], sem)
        out_cp.start()

        def colsum(ccol, _):
            def rstep(rr, accs):
                a0, a1, a2, a3 = accs
                cs = pl.ds(ccol * 16, 16)
                a0 = a0 + rows_v[rr * 4, cs]
                a1 = a1 + rows_v[rr * 4 + 1, cs]
                a2 = a2 + rows_v[rr * 4 + 2, cs]
                a3 = a3 + rows_v[rr * 4 + 3, cs]
                return (a0, a1, a2, a3)

            z = jnp.zeros((16,), jnp.float32)
            a0, a1, a2, a3 = lax.fori_loop(
                0, npp // 4, rstep, (z, z, z, z), unroll=2)
            acc_v[0, pl.ds(ccol * 16, 16)] = (a0 + a1) + (a2 + a3)
            return 0

        lax.fori_loop(0, D // 16, colsum, 0, unroll=4)
        pltpu.sync_copy(acc_v, psum_hbm.at[pl.ds(b * parts + p, 1)])
        out_cp.wait()

    return k(sal_bits, x2d)


def _p3_body(x_ref, wread_ref, sumall_ref, psum_ref, out_ref, cold_ref, v_scr):
    l = pl.program_id(1)

    @pl.when(l == 0)
    def _ctx():
        sk = jnp.sum(psum_ref[0], axis=0, keepdims=True)
        n_drop = x_ref.shape[1] * pl.num_programs(1) - ACTIVE
        cold = (sumall_ref[0] - sk) / float(n_drop)
        ctx = sk * (1.0 / ACTIVE) + cold
        v_scr[...] = lax.dot_general(
            ctx, wread_ref[...], (((1,), (1,)), ((), ())),
            preferred_element_type=jnp.float32)
        cold_ref[0] = cold

    out_ref[0] = x_ref[0] + v_scr[...]


def _pass3(x, w_read, sum_all, psum, BL):
    B, L, D = x.shape
    return pl.pallas_call(
        _p3_body,
        grid=(B, L // BL),
        in_specs=[
            pl.BlockSpec((1, BL, D), lambda b, l: (b, l, 0)),
            pl.BlockSpec((D, D), lambda b, l: (0, 0)),
            pl.BlockSpec((1, 1, D), lambda b, l: (b, 0, 0)),
            pl.BlockSpec((1, 8, D), lambda b, l: (b, 0, 0)),
        ],
        out_specs=[
            pl.BlockSpec((1, BL, D), lambda b, l: (b, l, 0)),
            pl.BlockSpec((1, 1, D), lambda b, l: (b, 0, 0)),
        ],
        out_shape=[
            jax.ShapeDtypeStruct((B, L, D), jnp.float32),
            jax.ShapeDtypeStruct((B, 1, D), jnp.float32),
        ],
        scratch_shapes=[pltpu.VMEM((1, D), jnp.float32)],
    )(x, w_read, sum_all, psum)


def kernel(x, w_sal, w_read):
    B, L, D = x.shape
    sal3, sum_all = _pass1(x, w_sal.T, 2048)
    sal_bits = lax.bitcast_convert_type(sal3.reshape(B * L), jnp.int32)
    act2d, psum2d = _sc_select_gather(sal_bits, x.reshape(B * L, D), B, L, D)
    x_out, cold3 = _pass3(x, w_read, sum_all, psum2d.reshape(B, 8, D), 2048)
    return x_out, act2d.reshape(B, ACTIVE, D), cold3.reshape(B, D)
